# baseline (speedup 1.0000x reference)
import jax
import jax.numpy as jnp
from jax import lax
from jax.experimental import pallas as pl
from jax.experimental.pallas import tpu as pltpu
from jax.experimental.pallas import tpu_sc as plsc

PATCH = 16
STRIDE = 2
C = 3
H = W = 512
OH = OW = (H - PATCH) // STRIDE + 1
NROWS = OH * OW
D = C * PATCH * PATCH
NSEG = C * PATCH

NC = 2
NS = 16
NW = NC * NS
JCH = 83


def _body(in1, in2, out1, out2, slab, obuf):
    wid = lax.axis_index("s") * NC + lax.axis_index("c")

    for in_ref, out_ref in ((in1, out1), (in2, out2)):

        @pl.loop(wid, OH, step=NW)
        def _block(i):
            for c in range(C):
                pltpu.sync_copy(
                    in_ref.at[pl.ds((c * H + STRIDE * i) * W, PATCH * W)],
                    slab.at[pl.ds(c * PATCH * W, PATCH * W)],
                )
            for k in range(OH // JCH):

                @pl.loop(0, JCH)
                def _row(jj):
                    j2 = STRIDE * (k * JCH + jj)
                    for s in range(NSEG):
                        obuf[pl.ds(jj * D + s * 16, 16)] = slab[
                            pl.ds(s * W + j2, 16)
                        ]

                pltpu.sync_copy(
                    obuf,
                    out_ref.at[pl.ds((i * OH + k * JCH) * D, JCH * D)],
                )


@jax.jit
def kernel(input_1, input_2):
    mesh = plsc.VectorSubcoreMesh(
        core_axis_name="c", subcore_axis_name="s", num_cores=NC, num_subcores=NS
    )
    out = jax.ShapeDtypeStruct((NROWS * D,), jnp.float32)
    p1, p2 = pl.kernel(
        _body,
        out_type=(out, out),
        mesh=mesh,
        scratch_types=[
            pltpu.VMEM((NSEG * W,), jnp.float32),
            pltpu.VMEM((JCH * D,), jnp.float32),
        ],
    )(input_1.reshape(-1), input_2.reshape(-1))
    return (p1.reshape(NROWS, D), p2.reshape(NROWS, D))


# speedup vs baseline: 1.7978x; 1.7978x over previous
import jax
import jax.numpy as jnp
from jax import lax
from jax.experimental import pallas as pl
from jax.experimental.pallas import tpu as pltpu
from jax.experimental.pallas import tpu_sc as plsc

PATCH = 16
STRIDE = 2
C = 3
H = W = 512
OH = OW = (H - PATCH) // STRIDE + 1
NROWS = OH * OW
D = C * PATCH * PATCH
NSEG = C * PATCH

NC = 2
NS = 16
NW = NC * NS
JCH = 32
NCH = -(-OH // JCH)
G = 12


def _body(in1, in2, out1, out2, slab, obufs, sems):
    wid = lax.axis_index("s") * NC + lax.axis_index("c")

    for in_ref, out_ref in ((in1, out1), (in2, out2)):

        @pl.loop(wid, OH, step=NW)
        def _block(i):
            for c in range(C):
                pltpu.sync_copy(
                    in_ref.at[pl.ds((c * H + STRIDE * i) * W, PATCH * W)],
                    slab.at[pl.ds(c * PATCH * W, PATCH * W)],
                )
            pending = {}
            for k in range(NCH):
                rows = min(JCH, OH - k * JCH)
                b = k % 2
                ob = obufs[b]
                if b in pending:
                    pending.pop(b).wait()

                @pl.loop(0, rows)
                def _row(jj):
                    j2 = STRIDE * (k * JCH + jj)
                    base = jj * D
                    for g0 in range(0, NSEG, G):
                        vals = [
                            slab[pl.ds(s * W + j2, 16)]
                            for s in range(g0, g0 + G)
                        ]
                        for t, s in enumerate(range(g0, g0 + G)):
                            ob[pl.ds(base + s * 16, 16)] = vals[t]

                pending[b] = pltpu.async_copy(
                    ob.at[pl.ds(0, rows * D)],
                    out_ref.at[pl.ds((i * OH + k * JCH) * D, rows * D)],
                    sems[b],
                )
            for h in pending.values():
                h.wait()


@jax.jit
def kernel(input_1, input_2):
    mesh = plsc.VectorSubcoreMesh(
        core_axis_name="c", subcore_axis_name="s", num_cores=NC, num_subcores=NS
    )
    out = jax.ShapeDtypeStruct((NROWS * D,), jnp.float32)
    p1, p2 = pl.kernel(
        _body,
        out_type=(out, out),
        mesh=mesh,
        scratch_types=[
            pltpu.VMEM((NSEG * W,), jnp.float32),
            tuple(pltpu.VMEM((JCH * D,), jnp.float32) for _ in range(2)),
            tuple(pltpu.SemaphoreType.DMA for _ in range(2)),
        ],
    )(input_1.reshape(-1), input_2.reshape(-1))
    return (p1.reshape(NROWS, D), p2.reshape(NROWS, D))


# speedup vs baseline: 2.0444x; 1.1372x over previous
import jax
import jax.numpy as jnp
from jax import lax
from jax.experimental import pallas as pl
from jax.experimental.pallas import tpu as pltpu
from jax.experimental.pallas import tpu_sc as plsc

PATCH = 16
STRIDE = 2
C = 3
H = W = 512
OH = OW = (H - PATCH) // STRIDE + 1
NROWS = OH * OW
D = C * PATCH * PATCH
NSEG = C * PATCH
SLAB = PATCH * W
CSLAB = C * SLAB

NC = 2
NS = 16
NW = NC * NS
NBLK = -(-OH // NW)
JCH = 32
NCH = -(-OH // JCH)
LAG = 10


def _emit_row(slab, soff, ob, jj, k):
    j2 = soff + STRIDE * (k * JCH + jj)
    base = jj * D
    vals = {}
    for s in range(NSEG):
        vals[s] = slab[pl.ds(j2 + s * W, 16)]
        if s >= LAG:
            ob[pl.ds(base + (s - LAG) * 16, 16)] = vals.pop(s - LAG)
    for s in range(NSEG - LAG, NSEG):
        ob[pl.ds(base + s * 16, 16)] = vals.pop(s)


def _slab_copy(in_ref, slab, i, soff, sem):
    return [
        pltpu.make_async_copy(
            in_ref.at[pl.ds((c * H + STRIDE * i) * W, SLAB)],
            slab.at[pl.ds(soff + c * SLAB, SLAB)],
            sem,
        )
        for c in range(C)
    ]


def _body(in1, in2, out1, out2, slab, ssem, obufs, osems):
    wid = lax.axis_index("s") * NC + lax.axis_index("c")

    for in_ref, out_ref in ((in1, out1), (in2, out2)):
        for cp in _slab_copy(in_ref, slab, wid, 0, ssem):
            cp.start()

        @pl.loop(0, NBLK)
        def _block(t):
            i = wid + t * NW
            soff = (t & 1) * CSLAB

            @pl.when(i < OH)
            def _():
                for cp in _slab_copy(in_ref, slab, i, soff, ssem):
                    cp.wait()
                nxt = i + NW

                @pl.when(nxt < OH)
                def _():
                    for cp in _slab_copy(in_ref, slab, nxt, CSLAB - soff, ssem):
                        cp.start()

                pending = {}
                for k in range(NCH):
                    rows = min(JCH, OH - k * JCH)
                    b = k % 2
                    ob = obufs[b]
                    if b in pending:
                        pending.pop(b).wait()

                    @pl.loop(0, rows, unroll=2)
                    def _row(jj):
                        _emit_row(slab, soff, ob, jj, k)

                    pending[b] = pltpu.async_copy(
                        ob.at[pl.ds(0, rows * D)],
                        out_ref.at[pl.ds((i * OH + k * JCH) * D, rows * D)],
                        osems[b],
                    )
                for h in pending.values():
                    h.wait()


@jax.jit
def kernel(input_1, input_2):
    mesh = plsc.VectorSubcoreMesh(
        core_axis_name="c", subcore_axis_name="s", num_cores=NC, num_subcores=NS
    )
    out = jax.ShapeDtypeStruct((NROWS * D,), jnp.float32)
    p1, p2 = pl.kernel(
        _body,
        out_type=(out, out),
        mesh=mesh,
        scratch_types=[
            pltpu.VMEM((2 * CSLAB,), jnp.float32),
            pltpu.SemaphoreType.DMA,
            tuple(pltpu.VMEM((JCH * D,), jnp.float32) for _ in range(2)),
            tuple(pltpu.SemaphoreType.DMA for _ in range(2)),
        ],
    )(input_1.reshape(-1), input_2.reshape(-1))
    return (p1.reshape(NROWS, D), p2.reshape(NROWS, D))
